# Initial kernel scaffold; baseline (speedup 1.0000x reference)
import jax
import jax.numpy as jnp
from jax.experimental import pallas as pl


def kernel(inputs, weight):
    raise NotImplementedError("write your pallas kernel here")


# speedup vs baseline: 2.9693x; 2.9693x over previous
import functools

import jax
import jax.numpy as jnp
from jax import lax
from jax.experimental import pallas as pl
from jax.experimental.pallas import tpu as pltpu
from jax.experimental.pallas import tpu_sc as plsc

VOCAB = 100000
DIM = 128
BATCH = 4096
HIST = 50

_info = plsc.get_sparse_core_info()
NC, NS = _info.num_cores, _info.num_subcores
NW = NC * NS

B_TOTAL = BATCH * HIST
B_PER_W = B_TOTAL // NW
CH = 128
NCHUNK = B_PER_W // CH


@functools.partial(
    pl.kernel,
    out_type=jax.ShapeDtypeStruct((B_TOTAL, DIM), jnp.float32),
    mesh=plsc.VectorSubcoreMesh(core_axis_name="c", subcore_axis_name="s"),
    scratch_types=[
        pltpu.VMEM((NCHUNK, CH), jnp.int32),
        pltpu.VMEM((CH, DIM), jnp.float32),
        pltpu.SemaphoreType.DMA,
    ],
)
def _gather_kernel(idx_hbm, table_hbm, out_hbm, idx_v, rows_v, gsem):
    wid = lax.axis_index("s") * NC + lax.axis_index("c")
    base = wid * B_PER_W
    pltpu.sync_copy(idx_hbm.at[wid], idx_v)

    @pl.loop(0, NCHUNK)
    def _chunk(j):
        pltpu.async_copy(table_hbm.at[idx_v.at[j]], rows_v, gsem).wait()
        pltpu.sync_copy(rows_v, out_hbm.at[pl.ds(base + j * CH, CH)])


def kernel(inputs, weight):
    idx = inputs.astype(jnp.int32).reshape(NW, NCHUNK, CH)
    out = _gather_kernel(idx, weight)
    return out.reshape(BATCH, HIST, DIM)


# speedup vs baseline: 3.3542x; 1.1296x over previous
import functools

import jax
import jax.numpy as jnp
from jax import lax
from jax.experimental import pallas as pl
from jax.experimental.pallas import tpu as pltpu
from jax.experimental.pallas import tpu_sc as plsc

VOCAB = 100000
DIM = 128
BATCH = 4096
HIST = 50

_info = plsc.get_sparse_core_info()
NC, NS = _info.num_cores, _info.num_subcores
NW = NC * NS

B_TOTAL = BATCH * HIST
B_PER_W = B_TOTAL // NW
CH = 128
NCHUNK = B_PER_W // CH
NBUF = 5
NGROUP = NCHUNK // NBUF


@functools.partial(
    pl.kernel,
    out_type=jax.ShapeDtypeStruct((B_TOTAL, DIM), jnp.float32),
    mesh=plsc.VectorSubcoreMesh(core_axis_name="c", subcore_axis_name="s"),
    scratch_types=[
        pltpu.VMEM((NCHUNK, CH), jnp.int32),
        pltpu.VMEM((NBUF, CH, DIM), jnp.float32),
        [pltpu.SemaphoreType.DMA] * NBUF,
        [pltpu.SemaphoreType.DMA] * NBUF,
    ],
)
def _gather_kernel(idx_hbm, table_hbm, out_hbm, idx_v, rows_v, gsems, ssems):
    wid = lax.axis_index("s") * NC + lax.axis_index("c")
    base = wid * B_PER_W
    pltpu.sync_copy(idx_hbm.at[wid], idx_v)

    def start_gather(j, b):
        pltpu.async_copy(table_hbm.at[idx_v.at[j]], rows_v.at[b], gsems[b])

    def wait_gather(b):
        pltpu.make_async_copy(
            table_hbm.at[idx_v.at[0]], rows_v.at[b], gsems[b]
        ).wait()

    def start_store(j, b):
        pltpu.async_copy(
            rows_v.at[b], out_hbm.at[pl.ds(base + j * CH, CH)], ssems[b]
        )

    def wait_store(b):
        pltpu.make_async_copy(
            rows_v.at[b], out_hbm.at[pl.ds(base, CH)], ssems[b]
        ).wait()

    def chunk(j, b, prefetch):
        wait_gather(b)
        start_store(j, b)
        if prefetch:
            bp = (b - 1) % NBUF
            wait_store(bp)
            start_gather(j + NBUF - 1, bp)

    for b in range(NBUF):
        start_gather(b, b)

    for b in range(NBUF):
        chunk(b, b, prefetch=(b > 0))

    @pl.loop(1, NGROUP - 1)
    def _group(g):
        j0 = g * NBUF
        for b in range(NBUF):
            chunk(j0 + b, b, prefetch=True)

    j0 = (NGROUP - 1) * NBUF
    for b in range(NBUF):
        chunk(j0 + b, b, prefetch=(b == 0))

    for b in range(NBUF):
        wait_store(b)


def kernel(inputs, weight):
    idx = inputs.astype(jnp.int32).reshape(NW, NCHUNK, CH)
    out = _gather_kernel(idx, weight)
    return out.reshape(BATCH, HIST, DIM)


# speedup vs baseline: 5.9469x; 1.7730x over previous
import functools

import jax
import jax.numpy as jnp
from jax import lax
from jax.experimental import pallas as pl
from jax.experimental.pallas import tpu as pltpu
from jax.experimental.pallas import tpu_sc as plsc

VOCAB = 100000
DIM = 128
BATCH = 4096
HIST = 50
HPAD = 64

_info = plsc.get_sparse_core_info()
NC, NS = _info.num_cores, _info.num_subcores
NW = NC * NS

B_PER_W = BATCH // NW
BPC = 2
NCHUNK = B_PER_W // BPC
NBUF = 4
NGROUP = NCHUNK // NBUF


@functools.partial(
    pl.kernel,
    out_type=jax.ShapeDtypeStruct((BATCH, HIST, DIM), jnp.float32),
    mesh=plsc.VectorSubcoreMesh(core_axis_name="c", subcore_axis_name="s"),
    compiler_params=pltpu.CompilerParams(use_tc_tiling_on_sc=True),
    scratch_types=[
        pltpu.VMEM((NCHUNK, BPC * HPAD), jnp.int32),
        pltpu.VMEM((NBUF, BPC, HIST, DIM), jnp.float32),
        [pltpu.SemaphoreType.DMA] * NBUF,
        [pltpu.SemaphoreType.DMA] * NBUF,
    ],
)
def _gather_kernel(idx_hbm, table_hbm, out_hbm, idx_v, rows_v, gsems, ssems):
    wid = lax.axis_index("s") * NC + lax.axis_index("c")
    base = wid * B_PER_W
    pltpu.sync_copy(idx_hbm.at[wid], idx_v)

    def start_gathers(j, b):
        for i in range(BPC):
            pltpu.async_copy(
                table_hbm.at[idx_v.at[j, pl.ds(i * HPAD, HIST)]],
                rows_v.at[b, i],
                gsems[b],
            )

    def wait_gathers(b):
        for i in range(BPC):
            pltpu.make_async_copy(
                table_hbm.at[idx_v.at[0, pl.ds(0, HIST)]],
                rows_v.at[b, i],
                gsems[b],
            ).wait()

    def start_store(j, b):
        pltpu.async_copy(
            rows_v.at[b], out_hbm.at[pl.ds(base + j * BPC, BPC)], ssems[b]
        )

    def wait_store(b):
        pltpu.make_async_copy(
            rows_v.at[b], out_hbm.at[pl.ds(base, BPC)], ssems[b]
        ).wait()

    def chunk(j, b, prefetch):
        wait_gathers(b)
        start_store(j, b)
        if prefetch:
            bp = (b - 1) % NBUF
            wait_store(bp)
            start_gathers(j + NBUF - 1, bp)

    for b in range(NBUF):
        start_gathers(b, b)

    for b in range(NBUF):
        chunk(b, b, prefetch=(b > 0))

    @pl.loop(1, NGROUP - 1)
    def _group(g):
        j0 = g * NBUF
        for b in range(NBUF):
            chunk(j0 + b, b, prefetch=True)

    j0 = (NGROUP - 1) * NBUF
    for b in range(NBUF):
        chunk(j0 + b, b, prefetch=(b == 0))

    for b in range(NBUF):
        wait_store(b)


def kernel(inputs, weight):
    idx = jnp.pad(inputs.astype(jnp.int32), ((0, 0), (0, HPAD - HIST)))
    idx = idx.reshape(NW, NCHUNK, BPC * HPAD)
    return _gather_kernel(idx, weight)


# speedup vs baseline: 10.4413x; 1.7558x over previous
import functools

import jax
import jax.numpy as jnp
from jax import lax
from jax.experimental import pallas as pl
from jax.experimental.pallas import tpu as pltpu
from jax.experimental.pallas import tpu_sc as plsc

VOCAB = 100000
DIM = 128
BATCH = 4096
HIST = 50

_info = plsc.get_sparse_core_info()
NC, NS = _info.num_cores, _info.num_subcores
NW = NC * NS

B_TOTAL = BATCH * HIST
B_PER_W = B_TOTAL // NW
CH = 128
NCHUNK = B_PER_W // CH
NBUF = 5
NGROUP = NCHUNK // NBUF


@functools.partial(
    pl.kernel,
    out_type=jax.ShapeDtypeStruct((B_TOTAL, DIM), jnp.float32),
    mesh=plsc.VectorSubcoreMesh(core_axis_name="c", subcore_axis_name="s"),
    compiler_params=pltpu.CompilerParams(use_tc_tiling_on_sc=True),
    scratch_types=[
        pltpu.VMEM((NCHUNK, CH), jnp.int32),
        pltpu.VMEM((NBUF, CH, DIM), jnp.float32),
        [pltpu.SemaphoreType.DMA] * NBUF,
        [pltpu.SemaphoreType.DMA] * NBUF,
    ],
)
def _gather_kernel(idx_hbm, table_hbm, out_hbm, idx_v, rows_v, gsems, ssems):
    wid = lax.axis_index("s") * NC + lax.axis_index("c")
    base = wid * B_PER_W
    pltpu.sync_copy(idx_hbm.at[wid], idx_v)

    def start_gather(j, b):
        pltpu.async_copy(table_hbm.at[idx_v.at[j]], rows_v.at[b], gsems[b])

    def wait_gather(b):
        pltpu.make_async_copy(
            table_hbm.at[idx_v.at[0]], rows_v.at[b], gsems[b]
        ).wait()

    def start_store(j, b):
        pltpu.async_copy(
            rows_v.at[b], out_hbm.at[pl.ds(base + j * CH, CH)], ssems[b]
        )

    def wait_store(b):
        pltpu.make_async_copy(
            rows_v.at[b], out_hbm.at[pl.ds(base, CH)], ssems[b]
        ).wait()

    def chunk(j, b, prefetch):
        wait_gather(b)
        start_store(j, b)
        if prefetch:
            bp = (b - 1) % NBUF
            wait_store(bp)
            start_gather(j + NBUF - 1, bp)

    for b in range(NBUF):
        start_gather(b, b)

    for b in range(NBUF):
        chunk(b, b, prefetch=(b > 0))

    @pl.loop(1, NGROUP - 1)
    def _group(g):
        j0 = g * NBUF
        for b in range(NBUF):
            chunk(j0 + b, b, prefetch=True)

    j0 = (NGROUP - 1) * NBUF
    for b in range(NBUF):
        chunk(j0 + b, b, prefetch=(b == 0))

    for b in range(NBUF):
        wait_store(b)


def kernel(inputs, weight):
    idx = inputs.astype(jnp.int32).T.reshape(NW, NCHUNK, CH)
    out = _gather_kernel(idx, weight)
    return out.reshape(HIST, BATCH, DIM).transpose(1, 0, 2)


# speedup vs baseline: 10.5206x; 1.0076x over previous
import functools

import jax
import jax.numpy as jnp
from jax import lax
from jax.experimental import pallas as pl
from jax.experimental.pallas import tpu as pltpu
from jax.experimental.pallas import tpu_sc as plsc

VOCAB = 100000
DIM = 128
BATCH = 4096
HIST = 50

_info = plsc.get_sparse_core_info()
NC, NS = _info.num_cores, _info.num_subcores
NW = NC * NS

B_TOTAL = BATCH * HIST
B_PER_W = B_TOTAL // NW
CH = 64
NCHUNK = B_PER_W // CH
NBUF = 10
NGROUP = NCHUNK // NBUF


@functools.partial(
    pl.kernel,
    out_type=jax.ShapeDtypeStruct((B_TOTAL, DIM), jnp.float32),
    mesh=plsc.VectorSubcoreMesh(core_axis_name="c", subcore_axis_name="s"),
    compiler_params=pltpu.CompilerParams(use_tc_tiling_on_sc=True),
    scratch_types=[
        pltpu.VMEM((NCHUNK, CH), jnp.int32),
        pltpu.VMEM((NBUF, CH, DIM), jnp.float32),
        [pltpu.SemaphoreType.DMA] * NBUF,
        [pltpu.SemaphoreType.DMA] * NBUF,
    ],
)
def _gather_kernel(idx_hbm, table_hbm, out_hbm, idx_v, rows_v, gsems, ssems):
    wid = lax.axis_index("s") * NC + lax.axis_index("c")
    base = wid * B_PER_W
    pltpu.sync_copy(idx_hbm.at[wid], idx_v)

    def start_gather(j, b):
        pltpu.async_copy(table_hbm.at[idx_v.at[j]], rows_v.at[b], gsems[b])

    def wait_gather(b):
        pltpu.make_async_copy(
            table_hbm.at[idx_v.at[0]], rows_v.at[b], gsems[b]
        ).wait()

    def start_store(j, b):
        pltpu.async_copy(
            rows_v.at[b], out_hbm.at[pl.ds(base + j * CH, CH)], ssems[b]
        )

    def wait_store(b):
        pltpu.make_async_copy(
            rows_v.at[b], out_hbm.at[pl.ds(base, CH)], ssems[b]
        ).wait()

    def chunk(j, b, prefetch):
        wait_gather(b)
        start_store(j, b)
        if prefetch:
            bp = (b - 1) % NBUF
            wait_store(bp)
            start_gather(j + NBUF - 1, bp)

    for b in range(NBUF):
        start_gather(b, b)

    for b in range(NBUF):
        chunk(b, b, prefetch=(b > 0))

    @pl.loop(1, NGROUP - 1)
    def _group(g):
        j0 = g * NBUF
        for b in range(NBUF):
            chunk(j0 + b, b, prefetch=True)

    j0 = (NGROUP - 1) * NBUF
    for b in range(NBUF):
        chunk(j0 + b, b, prefetch=(b == 0))

    for b in range(NBUF):
        wait_store(b)


def kernel(inputs, weight):
    idx = inputs.astype(jnp.int32).T.reshape(NW, NCHUNK, CH)
    out = _gather_kernel(idx, weight)
    return out.reshape(HIST, BATCH, DIM).transpose(1, 0, 2)
